# baseline (speedup 1.0000x reference)
import jax
import jax.numpy as jnp
from jax import lax
from jax.experimental import pallas as pl
from jax.experimental.pallas import tpu as pltpu
from jax.experimental.pallas import tpu_sc as plsc

N_NODES = 10000
HIDDEN = 256
HALF = HIDDEN // 2
N_EDGES = 160000

NSUB = 16
EDGES_PER_TILE = N_EDGES // NSUB
K = 80
NCHUNK = EDGES_PER_TILE // K
NPAD = 10240
ROWS_PER_TILE = NPAD // NSUB
ZROWS = K
NZ = ROWS_PER_TILE // ZROWS

G = 5000
NB = N_NODES // G


def _mm1_body(h_ref, w_ref, b_ref, p1_ref, p2_ref):
    h = h_ref[...]
    w = w_ref[...]
    p1 = jnp.dot(h, w[:HIDDEN], preferred_element_type=jnp.float32)
    p2 = (jnp.dot(h, w[HIDDEN:], preferred_element_type=jnp.float32)
          + b_ref[...])
    p1_ref[0] = p1[:, :HALF]
    p1_ref[1] = p1[:, HALF:]
    p2_ref[0] = p2[:, :HALF]
    p2_ref[1] = p2[:, HALF:]


def _mm1(h, w, b1):
    return pl.pallas_call(
        _mm1_body,
        grid=(NB,),
        in_specs=[
            pl.BlockSpec((G, HIDDEN), lambda i: (i, 0)),
            pl.BlockSpec((2 * HIDDEN, HIDDEN), lambda i: (0, 0)),
            pl.BlockSpec((1, HIDDEN), lambda i: (0, 0)),
        ],
        out_specs=[
            pl.BlockSpec((2, G, HALF), lambda i: (0, i, 0)),
            pl.BlockSpec((2, G, HALF), lambda i: (0, i, 0)),
        ],
        out_shape=[
            jax.ShapeDtypeStruct((2, N_NODES, HALF), jnp.float32),
            jax.ShapeDtypeStruct((2, N_NODES, HALF), jnp.float32),
        ],
    )(h, w, b1)


def _upd(h_ref, agga_ref, aggb_ref, w_ref, b_ref, out_ref):
    h = h_ref[...]
    w = w_ref[...]
    u = (jnp.dot(h, w[:HIDDEN], preferred_element_type=jnp.float32)
         + jnp.dot(agga_ref[0], w[HIDDEN:HIDDEN + HALF],
                   preferred_element_type=jnp.float32)
         + jnp.dot(aggb_ref[0], w[HIDDEN + HALF:],
                   preferred_element_type=jnp.float32)
         + b_ref[...])
    out_ref[...] = h + jnp.maximum(u, 0.0)


def _mm2_body(h_ref, agga_ref, aggb_ref, w_ref, b_ref, out_ref):
    _upd(h_ref, agga_ref, aggb_ref, w_ref, b_ref, out_ref)


def _mm2(h, w, agg, b1):
    return pl.pallas_call(
        _mm2_body,
        grid=(NB,),
        in_specs=[
            pl.BlockSpec((G, HIDDEN), lambda i: (i, 0)),
            pl.BlockSpec((1, G, HALF), lambda i: (0, i, 0)),
            pl.BlockSpec((1, G, HALF), lambda i: (1, i, 0)),
            pl.BlockSpec((2 * HIDDEN, HIDDEN), lambda i: (0, 0)),
            pl.BlockSpec((1, HIDDEN), lambda i: (0, 0)),
        ],
        out_specs=pl.BlockSpec((G, HIDDEN), lambda i: (i, 0)),
        out_shape=jax.ShapeDtypeStruct((N_NODES, HIDDEN), jnp.float32),
    )(h, agg, agg, w, b1)


def _fused_body(h_ref, agga_ref, aggb_ref, wu_ref, bu_ref, wm_ref, bm_ref,
                hn_ref, p1_ref, p2_ref):
    _upd(h_ref, agga_ref, aggb_ref, wu_ref, bu_ref, hn_ref)
    _mm1_body(hn_ref, wm_ref, bm_ref, p1_ref, p2_ref)


def _fused(h, agg, wu, bu, wm, bm):
    return pl.pallas_call(
        _fused_body,
        grid=(NB,),
        in_specs=[
            pl.BlockSpec((G, HIDDEN), lambda i: (i, 0)),
            pl.BlockSpec((1, G, HALF), lambda i: (0, i, 0)),
            pl.BlockSpec((1, G, HALF), lambda i: (1, i, 0)),
            pl.BlockSpec((2 * HIDDEN, HIDDEN), lambda i: (0, 0)),
            pl.BlockSpec((1, HIDDEN), lambda i: (0, 0)),
            pl.BlockSpec((2 * HIDDEN, HIDDEN), lambda i: (0, 0)),
            pl.BlockSpec((1, HIDDEN), lambda i: (0, 0)),
        ],
        out_specs=[
            pl.BlockSpec((G, HIDDEN), lambda i: (i, 0)),
            pl.BlockSpec((2, G, HALF), lambda i: (0, i, 0)),
            pl.BlockSpec((2, G, HALF), lambda i: (0, i, 0)),
        ],
        out_shape=[
            jax.ShapeDtypeStruct((N_NODES, HIDDEN), jnp.float32),
            jax.ShapeDtypeStruct((2, N_NODES, HALF), jnp.float32),
            jax.ShapeDtypeStruct((2, N_NODES, HALF), jnp.float32),
        ],
    )(h, agg, agg, wu, bu, wm, bm)


def _edge_body(p1_hbm, p2_hbm, iso_hbm, ido_hbm, idr_hbm, agg_hbm,
               iso0, iso1, ido0, ido1, idr0, idr1, sidr0, sidr1,
               ra0, ra1, rb0, rb1, acc,
               g1s0, g1s1, g2s0, g2s1, is0, is1, ss0, ss1):
    c = lax.axis_index("c")
    s = lax.axis_index("s")
    zero = jnp.zeros((16,), jnp.float32)

    iso = (iso0, iso1)
    ido = (ido0, ido1)
    idr = (idr0, idr1)
    sidr = (sidr0, sidr1)
    ra = (ra0, ra1)
    rb = (rb0, rb1)
    g1s = (g1s0, g1s1)
    g2s = (g2s0, g2s1)
    isem = (is0, is1)
    ssem = (ss0, ss1)

    def zrow(i, carry):
        for j in range(HALF // 16):
            ra0[i, pl.ds(j * 16, 16)] = zero
        return carry

    lax.fori_loop(0, ZROWS, zrow, 0)
    row0 = s * ROWS_PER_TILE
    for k in range(NZ):
        pltpu.sync_copy(ra0, acc.at[pl.ds(row0 + k * ZROWS, ZROWS)])
    plsc.subcore_barrier()

    base_o = c * N_EDGES + s * EDGES_PER_TILE
    base_r = s * EDGES_PER_TILE

    def idx_start(ci, b):
        e0 = ci * K
        pltpu.async_copy(iso_hbm.at[pl.ds(base_o + e0, K)], iso[b], isem[b])
        pltpu.async_copy(ido_hbm.at[pl.ds(base_o + e0, K)], ido[b], isem[b])
        pltpu.async_copy(idr_hbm.at[pl.ds(base_r + e0, K)], idr[b], isem[b])

    def idx_wait_and_gather(ci, b):
        e0 = ci * K
        pltpu.make_async_copy(iso_hbm.at[pl.ds(base_o + e0, K)], iso[b],
                              isem[b]).wait()
        pltpu.make_async_copy(ido_hbm.at[pl.ds(base_o + e0, K)], ido[b],
                              isem[b]).wait()
        pltpu.make_async_copy(idr_hbm.at[pl.ds(base_r + e0, K)], idr[b],
                              isem[b]).wait()

        @pl.when(ci >= 2)
        def _():
            wait_scatter(b)
        pltpu.async_copy(p1_hbm.at[iso[b]], ra[b], g1s[b])
        pltpu.async_copy(p2_hbm.at[ido[b]], rb[b], g2s[b])

    def wait_gathers(b):
        pltpu.make_async_copy(p1_hbm.at[iso[b]], ra[b], g1s[b]).wait()
        pltpu.make_async_copy(p2_hbm.at[ido[b]], rb[b], g2s[b]).wait()

    def compute(b):
        def erow(e, ecarry):
            for j in range(HALF // 16):
                sl = pl.ds(j * 16, 16)
                ra[b][e, sl] = jnp.maximum(ra[b][e, sl] + rb[b][e, sl], 0.0)
            return ecarry

        lax.fori_loop(0, K, erow, 0)

    def start_scatter(b):
        for i in range(K // 16):
            sl = pl.ds(i * 16, 16)
            sidr[b][sl] = idr[b][sl]
        pltpu.async_copy(ra[b], acc.at[sidr[b]], ssem[b], add=True)

    def wait_scatter(b):
        pltpu.make_async_copy(ra[b], acc.at[sidr[b]], ssem[b]).wait()

    NPAIR = (NCHUNK - 1) // 2
    idx_start(0, 0)
    idx_wait_and_gather(0, 0)
    idx_start(1, 1)

    def pair(g, carry):
        ci = 2 * g
        wait_gathers(0)
        idx_wait_and_gather(ci + 1, 1)
        compute(0)
        start_scatter(0)
        idx_start(ci + 2, 0)
        wait_gathers(1)
        idx_wait_and_gather(ci + 2, 0)
        compute(1)
        start_scatter(1)

        @pl.when(g < NPAIR - 1)
        def _():
            idx_start(ci + 3, 1)
        return carry

    lax.fori_loop(0, NPAIR, pair, 0)

    wait_gathers(0)
    compute(0)
    start_scatter(0)
    wait_scatter(0)
    wait_scatter(1)
    plsc.subcore_barrier()

    for k in range(NZ):
        pltpu.sync_copy(acc.at[pl.ds(row0 + k * ZROWS, ZROWS)],
                        agg_hbm.at[c, pl.ds(row0 + k * ZROWS, ZROWS)])


_edge_call = pl.kernel(
    _edge_body,
    mesh=plsc.VectorSubcoreMesh(core_axis_name="c", subcore_axis_name="s"),
    out_type=jax.ShapeDtypeStruct((2, NPAD, HALF), jnp.float32),
    scratch_types=(
        [pltpu.VMEM((K,), jnp.int32)] * 8
        + [pltpu.VMEM((K, HALF), jnp.float32)] * 4
        + [pltpu.VMEM_SHARED((NPAD, HALF), jnp.float32)]
        + [pltpu.SemaphoreType.DMA] * 8
    ),
)


def kernel(x, edge_index, W_msg, b_msg, W_upd, b_upd):
    B, N, C = x.shape
    h = x.reshape(B * N, C)
    src = edge_index[0].astype(jnp.int32)
    dst = edge_index[1].astype(jnp.int32)
    iso_all = jnp.concatenate([src, src + N_NODES])
    ido_all = jnp.concatenate([dst, dst + N_NODES])
    depth = W_msg.shape[0]
    p1, p2 = _mm1(h, W_msg[0], b_msg[0].reshape(1, HIDDEN))
    for l in range(depth):
        agg = _edge_call(p1.reshape(2 * N_NODES, HALF),
                         p2.reshape(2 * N_NODES, HALF),
                         iso_all, ido_all, dst)
        if l < depth - 1:
            h, p1, p2 = _fused(h, agg, W_upd[l], b_upd[l].reshape(1, HIDDEN),
                               W_msg[l + 1], b_msg[l + 1].reshape(1, HIDDEN))
        else:
            h = _mm2(h, W_upd[l], agg, b_upd[l].reshape(1, HIDDEN))
    return h.reshape(B, N, C)
